# baseline (speedup 1.0000x reference)
import functools

import jax
import jax.numpy as jnp
from jax import lax
from jax.experimental import pallas as pl
from jax.experimental.pallas import tpu as pltpu
from jax.experimental.pallas import tpu_sc as plsc

VOCAB = 1000
N_EMBED = 16
TOT = 20480

NUM_CORES = 2
NUM_SUBCORES = 16
NW = NUM_CORES * NUM_SUBCORES
TPT = TOT // NW
NT = TOT // 128


def _embed_body(tab_hbm, idx_hbm, out_hbm, idx_v, rows_v, sem):
    wid = lax.axis_index("s") * NUM_CORES + lax.axis_index("c")
    base = wid * TPT
    pltpu.sync_copy(idx_hbm.at[pl.ds(base, TPT)], idx_v)
    pltpu.async_copy(tab_hbm.at[idx_v], rows_v, sem).wait()
    pltpu.sync_copy(rows_v, out_hbm.at[pl.ds(base, TPT)])


def _embed(table, idx_flat):
    mesh = plsc.VectorSubcoreMesh(core_axis_name="c", subcore_axis_name="s")
    k = functools.partial(
        pl.kernel,
        out_type=jax.ShapeDtypeStruct((TOT, N_EMBED), jnp.float32),
        mesh=mesh,
        scratch_types=[
            pltpu.VMEM((TPT,), jnp.int32),
            pltpu.VMEM((TPT, N_EMBED), jnp.float32),
            pltpu.SemaphoreType.DMA,
        ],
        compiler_params=pltpu.CompilerParams(
            use_tc_tiling_on_sc=False, needs_layout_passes=False
        ),
    )(_embed_body)
    return k(table, idx_flat)


def _proj_body(w_ref, b_ref, e_ref, o_ref):
    res = (
        lax.dot_general(
            w_ref[...],
            e_ref[...],
            (((0,), (1,)), ((), ())),
            preferred_element_type=jnp.float32,
        )
        + b_ref[...]
    )
    o_ref[...] = res.reshape(VOCAB // 8, 8, 1, 128).transpose(0, 2, 1, 3)


def _proj(w, b, embed):
    return pl.pallas_call(
        _proj_body,
        grid=(NT,),
        in_specs=[
            pl.BlockSpec((N_EMBED, VOCAB), lambda g: (0, 0)),
            pl.BlockSpec((VOCAB, 1), lambda g: (0, 0)),
            pl.BlockSpec((128, N_EMBED), lambda g: (g, 0)),
        ],
        out_specs=pl.BlockSpec((VOCAB // 8, 1, 8, 128), lambda g: (0, g, 0, 0)),
        out_shape=jax.ShapeDtypeStruct((VOCAB // 8, NT, 8, 128), jnp.float32),
    )(w, b.reshape(VOCAB, 1), embed)


def kernel(idx, token_embedding_table, lm_head_w, lm_head_b):
    idx_flat = idx.reshape(-1).astype(jnp.int32)
    embed = _embed(token_embedding_table, idx_flat)
    out4 = _proj(lm_head_w, lm_head_b, embed)
    return out4.transpose(1, 3, 0, 2).reshape(TOT, VOCAB)


# speedup vs baseline: 1.0778x; 1.0778x over previous
import functools

import jax
import jax.numpy as jnp
from jax import lax
from jax.experimental import pallas as pl
from jax.experimental.pallas import tpu as pltpu
from jax.experimental.pallas import tpu_sc as plsc

VOCAB = 1000
N_EMBED = 16
TOT = 20480

NUM_CORES = 2
NUM_SUBCORES = 16
NW = NUM_CORES * NUM_SUBCORES
TPT = TOT // NW
NT = TOT // 128


def _embed_body(tab_hbm, idx_hbm, out_hbm, idx_v, rows_v, sem):
    wid = lax.axis_index("s") * NUM_CORES + lax.axis_index("c")
    base = wid * TPT
    pltpu.sync_copy(idx_hbm.at[pl.ds(base, TPT)], idx_v)
    pltpu.async_copy(tab_hbm.at[idx_v], rows_v, sem).wait()
    pltpu.sync_copy(rows_v, out_hbm.at[pl.ds(base, TPT)])


def _embed(table, idx_flat):
    mesh = plsc.VectorSubcoreMesh(core_axis_name="c", subcore_axis_name="s")
    k = functools.partial(
        pl.kernel,
        out_type=jax.ShapeDtypeStruct((TOT, N_EMBED), jnp.float32),
        mesh=mesh,
        scratch_types=[
            pltpu.VMEM((TPT,), jnp.int32),
            pltpu.VMEM((TPT, N_EMBED), jnp.float32),
            pltpu.SemaphoreType.DMA,
        ],
        compiler_params=pltpu.CompilerParams(
            use_tc_tiling_on_sc=False, needs_layout_passes=False
        ),
    )(_embed_body)
    return k(table, idx_flat)


def _proj_body(w_ref, b_ref, e_ref, o_ref):
    res = (
        lax.dot_general(
            w_ref[...],
            e_ref[...],
            (((0,), (1,)), ((), ())),
            preferred_element_type=jnp.float32,
        )
        + b_ref[...]
    )
    o_ref[...] = res.reshape(VOCAB // 8, 1, 8, 128)


def _proj(w, b, embed):
    return pl.pallas_call(
        _proj_body,
        grid=(NT,),
        in_specs=[
            pl.BlockSpec((N_EMBED, VOCAB), lambda g: (0, 0)),
            pl.BlockSpec((VOCAB, 1), lambda g: (0, 0)),
            pl.BlockSpec((128, N_EMBED), lambda g: (g, 0)),
        ],
        out_specs=pl.BlockSpec((VOCAB // 8, 1, 8, 128), lambda g: (0, g, 0, 0)),
        out_shape=jax.ShapeDtypeStruct((VOCAB // 8, NT, 8, 128), jnp.float32),
    )(w, b.reshape(VOCAB, 1), embed)


def kernel(idx, token_embedding_table, lm_head_w, lm_head_b):
    idx_flat = idx.reshape(-1).astype(jnp.int32)
    embed = _embed(token_embedding_table, idx_flat)
    out4 = _proj(lm_head_w, lm_head_b, embed)
    return out4.transpose(1, 3, 0, 2).reshape(TOT, VOCAB)


# speedup vs baseline: 2.2083x; 2.0490x over previous
import functools

import jax
import jax.numpy as jnp
from jax import lax
from jax.experimental import pallas as pl
from jax.experimental.pallas import tpu as pltpu
from jax.experimental.pallas import tpu_sc as plsc

VOCAB = 1000
VOCAB_PAD = 1024
N_EMBED = 16
TOT = 20480

NUM_CORES = 2
NUM_SUBCORES = 16
NW = NUM_CORES * NUM_SUBCORES
VPT = VOCAB_PAD // NW
TAIL_V = VOCAB - (NW - 1) * VPT
NBLK = 1024
NCHUNK = NBLK // 16
NBLOCKS = TOT // NBLK


def _logits_table_body(t_ref, w_ref, b_ref, o_ref):
    o_ref[pl.ds(0, VOCAB), pl.ds(0, VOCAB)] = (
        lax.dot_general(
            w_ref[...],
            t_ref[...],
            (((0,), (1,)), ((), ())),
            preferred_element_type=jnp.float32,
        )
        + b_ref[...]
    )


def _logits_table(table, w, b):
    return pl.pallas_call(
        _logits_table_body,
        out_shape=jax.ShapeDtypeStruct((VOCAB_PAD, VOCAB_PAD), jnp.float32),
    )(table, w, b.reshape(VOCAB, 1))


def _tgather_body(lt_hbm, idx_hbm, out4_hbm, idx_v, slice_v, ob0, ob1, sem0, sem1):
    wid = lax.axis_index("s") * NUM_CORES + lax.axis_index("c")
    v0 = wid * VPT
    tr0 = wid * (VPT // 8)
    full = v0 + VPT <= VOCAB

    pltpu.sync_copy(idx_hbm, idx_v)
    pltpu.sync_copy(lt_hbm.at[pl.ds(tr0 * 8 * VOCAB_PAD, VPT * VOCAB_PAD)], slice_v)

    def mk_dma(ob, sem, b):
        tc0 = b * (NBLK // 128)
        dst_full = out4_hbm.at[pl.ds(tr0, VPT // 8), pl.ds(tc0, NBLK // 128)]
        dst_part = out4_hbm.at[pl.ds(tr0, TAIL_V // 8), pl.ds(tc0, NBLK // 128)]
        return (
            pltpu.make_async_copy(ob, dst_full, sem),
            pltpu.make_async_copy(ob.at[pl.ds(0, TAIL_V // 8)], dst_part, sem),
        )

    def process_block(b, ob, sem):
        @pl.when(b >= 2)
        def _():
            cf, cp = mk_dma(ob, sem, b - 2)
            lax.cond(full, cf.wait, cp.wait)

        @plsc.parallel_loop(0, NCHUNK)
        def _fill(j):
            n0 = b * NBLK + j * 16
            ivec = idx_v[pl.ds(n0, 16)]
            ihi = lax.shift_right_logical(ivec, 7)
            ilo = lax.bitwise_and(ivec, 127)
            uoff = ihi * 1024 + ilo
            tc = j // 8
            off = (j % 8) * 16
            for dv in range(VPT):
                dvo = (dv // 8) * 8 * VOCAB_PAD + (dv % 8) * 128
                ob[dv // 8, tc, dv % 8, pl.ds(off, 16)] = plsc.load_gather(
                    slice_v, [uoff + jnp.int32(dvo)]
                )
        cf, cp = mk_dma(ob, sem, b)
        lax.cond(full, cf.start, cp.start)

    def step(bb, _):
        process_block(2 * bb, ob0, sem0)
        process_block(2 * bb + 1, ob1, sem1)
        return 0

    lax.fori_loop(0, NBLOCKS // 2, step, 0)
    cf, cp = mk_dma(ob0, sem0, NBLOCKS - 2)
    lax.cond(full, cf.wait, cp.wait)
    cf, cp = mk_dma(ob1, sem1, NBLOCKS - 1)
    lax.cond(full, cf.wait, cp.wait)


def _tgather(lt, idx_flat):
    mesh = plsc.VectorSubcoreMesh(core_axis_name="c", subcore_axis_name="s")
    k = functools.partial(
        pl.kernel,
        out_type=jax.ShapeDtypeStruct((VOCAB // 8, TOT // 128, 8, 128), jnp.float32),
        mesh=mesh,
        scratch_types=[
            pltpu.VMEM((TOT,), jnp.int32),
            pltpu.VMEM((VPT * VOCAB_PAD,), jnp.float32),
            pltpu.VMEM((VPT // 8, NBLK // 128, 8, 128), jnp.float32),
            pltpu.VMEM((VPT // 8, NBLK // 128, 8, 128), jnp.float32),
            pltpu.SemaphoreType.DMA,
            pltpu.SemaphoreType.DMA,
        ],
        compiler_params=pltpu.CompilerParams(
            use_tc_tiling_on_sc=False, needs_layout_passes=False
        ),
    )(_tgather_body)
    return k(lt, idx_flat)


def kernel(idx, token_embedding_table, lm_head_w, lm_head_b):
    lt = _logits_table(token_embedding_table, lm_head_w, lm_head_b)
    lt4 = lt.reshape(VOCAB_PAD // 8, 8, VOCAB_PAD // 128, 128).transpose(
        0, 2, 1, 3
    ).reshape(-1)
    idx_flat = idx.reshape(-1).astype(jnp.int32)
    out4 = _tgather(lt4, idx_flat)
    return out4.transpose(1, 3, 0, 2).reshape(TOT, VOCAB)
